# baseline (speedup 1.0000x reference)
import jax
import jax.numpy as jnp
from jax.experimental import pallas as pl
from jax.experimental.pallas import tpu as pltpu

_ROWS = 16384
_STRIPES = 32
_R = _ROWS // _STRIPES


def _copy_body(v_ref, o_ref, sems):
    for i in range(_STRIPES):
        pltpu.make_async_copy(
            v_ref.at[pl.ds(i * _R, _R), :],
            o_ref.at[pl.ds(i * _R, _R), :],
            sems.at[i],
        ).start()
    for i in range(_STRIPES):
        pltpu.make_async_copy(
            v_ref.at[pl.ds(i * _R, _R), :],
            o_ref.at[pl.ds(i * _R, _R), :],
            sems.at[i],
        ).wait()


def kernel(value, index, cache):
    del index, cache
    return pl.pallas_call(
        _copy_body,
        out_shape=jax.ShapeDtypeStruct(value.shape, value.dtype),
        in_specs=[pl.BlockSpec(memory_space=pl.ANY)],
        out_specs=pl.BlockSpec(memory_space=pl.ANY),
        scratch_shapes=[pltpu.SemaphoreType.DMA((_STRIPES,))],
    )(value)
